# baseline (speedup 1.0000x reference)
import functools

import jax
import jax.numpy as jnp
from jax import lax
from jax.experimental import pallas as pl
from jax.experimental.pallas import tpu as pltpu
from jax.experimental.pallas import tpu_sc as plsc

N = 10000
NP = 10240
E = 160000
DN = 256
H = 512
NC = 2
NS = 16
NW = NC * NS
EW = E // NW
ES = E // NS
ROWS = NP // NS

@functools.lru_cache(maxsize=None)
def _sc_kernels():
    mesh = plsc.VectorSubcoreMesh(
        core_axis_name="c", subcore_axis_name="s", num_cores=NC, num_subcores=NS
    )

    @functools.partial(
        pl.kernel,
        out_type=jax.ShapeDtypeStruct((NC * NP,), jnp.float32),
        mesh=mesh,
        scratch_types=[
            pltpu.VMEM((40, 125), jnp.int32),
            pltpu.VMEM((125,), jnp.float32),
            pltpu.VMEM_SHARED((NP,), jnp.float32),
        ],
    )
    def degree_k(dst_r, ones, zeros1, out, idx_v, ones_v, acc):
        c = lax.axis_index("c")
        s = lax.axis_index("s")
        wid = s * NC + c
        pltpu.sync_copy(zeros1, acc.at[pl.ds(s * ROWS, ROWS)])
        pltpu.sync_copy(dst_r.at[wid], idx_v)
        pltpu.sync_copy(ones, ones_v)
        plsc.subcore_barrier()

        @pl.loop(0, 40)
        def _(j):
            pltpu.sync_copy(ones_v, acc.at[idx_v.at[j]], add=True)

        plsc.subcore_barrier()
        pltpu.sync_copy(acc.at[pl.ds(s * ROWS, ROWS)],
                        out.at[pl.ds(c * NP + s * ROWS, ROWS)])


    @functools.partial(
        pl.kernel,
        out_type=[jax.ShapeDtypeStruct((NP, 128), jnp.float32) for _ in range(4)],
        mesh=mesh,
        scratch_types=[
            pltpu.VMEM((40, 125), jnp.int32),
            pltpu.VMEM((40, 125), jnp.int32),
            pltpu.VMEM((125, 128), jnp.float32),
            pltpu.VMEM((125, 128), jnp.float32),
            pltpu.VMEM_SHARED((NP, 128), jnp.float32),
            pltpu.SemaphoreType.DMA,
            pltpu.SemaphoreType.DMA,
        ],
    )
    def agg_k(hs0, hs1, hs2, hs3, src16, dst16, zeros2, a0, a1, a2, a3,
                idxs, idxd, buf0, buf1, acc, sem0, sem1):
        c = lax.axis_index("c")
        s = lax.axis_index("s")

        def do_chunk(hs, agg):
            pltpu.sync_copy(zeros2, acc.at[pl.ds(s * ROWS, ROWS)])
            plsc.subcore_barrier()
            for half in range(2):
                pltpu.sync_copy(src16.at[s, half], idxs)
                pltpu.sync_copy(dst16.at[s, half], idxd)
                pltpu.async_copy(hs.at[idxs.at[0]], buf0, sem0)

                @pl.loop(0, 20)
                def _(t):
                    j0 = t * 2
                    pltpu.make_async_copy(hs.at[idxs.at[j0]], buf0, sem0).wait()
                    pltpu.async_copy(hs.at[idxs.at[j0 + 1]], buf1, sem1)
                    pltpu.sync_copy(buf0, acc.at[idxd.at[j0]], add=True)
                    pltpu.make_async_copy(hs.at[idxs.at[j0 + 1]], buf1,
                                          sem1).wait()

                    @pl.when(t < 19)
                    def _():
                        pltpu.async_copy(hs.at[idxs.at[j0 + 2]], buf0, sem0)

                    pltpu.sync_copy(buf1, acc.at[idxd.at[j0 + 1]], add=True)

            plsc.subcore_barrier()
            pltpu.sync_copy(acc.at[pl.ds(s * ROWS, ROWS)],
                            agg.at[pl.ds(s * ROWS, ROWS)])

        @pl.when(c == 0)
        def _():
            do_chunk(hs0, a0)
            do_chunk(hs1, a1)

        @pl.when(c == 1)
        def _():
            do_chunk(hs2, a2)
            do_chunk(hs3, a3)


    def make_edge_gather(g_lo, g_hi):
        cnt = g_hi - g_lo
        npairs = cnt // 2

        @functools.partial(
            pl.kernel,
            out_type=jax.ShapeDtypeStruct((NW * cnt * 40, H), jnp.float32),
            mesh=mesh,
            scratch_types=[
                pltpu.VMEM((125, 40), jnp.int32),
                pltpu.VMEM((125, 40), jnp.int32),
                pltpu.VMEM((2, 40, H), jnp.float32),
                pltpu.VMEM((2, 40, H), jnp.float32),
                pltpu.SemaphoreType.DMA,
                pltpu.SemaphoreType.DMA,
                pltpu.SemaphoreType.DMA,
                pltpu.SemaphoreType.DMA,
            ],
        )
        def edge_gather_k(A, B, src32, dst32, G, idxs, idxd, bufa, bufb,
                          sa0, sb0, sa1, sb1):
            c = lax.axis_index("c")
            s = lax.axis_index("s")
            wid = s * NC + c
            pltpu.sync_copy(src32.at[wid], idxs)
            pltpu.sync_copy(dst32.at[wid], idxd)
            obase = wid * cnt * 40

            def fire(j, p, sa, sb):
                pltpu.async_copy(A.at[idxs.at[j]], bufa.at[p], sa)
                pltpu.async_copy(B.at[idxd.at[j]], bufb.at[p], sb)

            def drain(j, p, sa, sb):
                pltpu.make_async_copy(A.at[idxs.at[j]], bufa.at[p], sa).wait()
                pltpu.make_async_copy(B.at[idxd.at[j]], bufb.at[p], sb).wait()

            def addout(j, p):
                @pl.loop(0, 40)
                def _(r):
                    for k in range(H // 16):
                        sl = pl.ds(k * 16, 16)
                        bufa[p, r, sl] = bufa[p, r, sl] + bufb[p, r, sl]

                pltpu.sync_copy(bufa.at[p],
                                G.at[pl.ds(obase + (j - g_lo) * 40, 40)])

            fire(g_lo, 0, sa0, sb0)

            @pl.loop(0, npairs)
            def _(t):
                j0 = g_lo + t * 2
                drain(j0, 0, sa0, sb0)
                fire(j0 + 1, 1, sa1, sb1)
                addout(j0, 0)
                drain(j0 + 1, 1, sa1, sb1)

                @pl.when(j0 + 2 < g_hi)
                def _():
                    fire(j0 + 2, 0, sa0, sb0)

                addout(j0 + 1, 1)

            if cnt % 2:
                drain(g_hi - 1, 0, sa0, sb0)
                addout(g_hi - 1, 0)

        return edge_gather_k

    return degree_k, agg_k, make_edge_gather(0, 64), make_edge_gather(64, 125)


def _degree_sc(*args):
    return _sc_kernels()[0](*args)


def _agg_sc(*args):
    return _sc_kernels()[1](*args)


def _edge_gather_sc(half, *args):
    return _sc_kernels()[2 + half](*args)


def _mm1_tc(x_ref, w_ref, pt_ref, h_ref, o0, o1, o2, o3):
    h = jnp.dot(x_ref[...], w_ref[...], preferred_element_type=jnp.float32)
    h_ref[...] = h
    deg = 1.0 + pt_ref[:, 0:1] + pt_ref[:, 1:2]
    hs = h * lax.rsqrt(deg)
    o0[...] = hs[:, 0:128]
    o1[...] = hs[:, 128:256]
    o2[...] = hs[:, 256:384]
    o3[...] = hs[:, 384:512]


def _layer2_tc(a0, a1, a2, a3, h1_ref, pt_ref, b1_ref, w2_ref,
               h2_ref, q0, q1, q2, q3):
    agg = jnp.concatenate([a0[...], a1[...], a2[...], a3[...]], axis=1)
    deg = 1.0 + pt_ref[:, 0:1] + pt_ref[:, 1:2]
    dinv = lax.rsqrt(deg)
    z1 = jnp.maximum(agg * dinv + h1_ref[...] * (1.0 / deg) + b1_ref[...], 0.0)
    h2 = jnp.dot(z1, w2_ref[...], preferred_element_type=jnp.float32)
    h2_ref[...] = h2
    hs = h2 * dinv
    q0[...] = hs[:, 0:128]
    q1[...] = hs[:, 128:256]
    q2[...] = hs[:, 256:384]
    q3[...] = hs[:, 384:512]


def _tables_tc(a0, a1, a2, a3, h2_ref, pt_ref, b2_ref, wa_ref, wb_ref,
               A_ref, B_ref):
    agg = jnp.concatenate([a0[...], a1[...], a2[...], a3[...]], axis=1)
    deg = 1.0 + pt_ref[:, 0:1] + pt_ref[:, 1:2]
    z2 = agg * lax.rsqrt(deg) + h2_ref[...] * (1.0 / deg) + b2_ref[...]
    A_ref[...] = jnp.dot(z2, wa_ref[...], preferred_element_type=jnp.float32)
    B_ref[...] = jnp.dot(z2, wb_ref[...], preferred_element_type=jnp.float32)


def _edge_mlp_tc(g_ref, ea_ref, wme_ref, bm1_ref, wm2_ref, bm2_ref, out_ref):
    cc = jnp.dot(ea_ref[0, 0], wme_ref[...], preferred_element_type=jnp.float32)
    t = jnp.maximum(g_ref[0, 0] + cc + bm1_ref[...], 0.0)
    out_ref[0, 0] = (
        jnp.dot(t, wm2_ref[...], preferred_element_type=jnp.float32)
        + bm2_ref[...]
    )


def _f32(*shape):
    return jax.ShapeDtypeStruct(shape, jnp.float32)


def kernel(x, edge_index, edge_attr, W1, b1, W2, b2, Wm1, bm1, Wm2, bm2):
    src = edge_index[0]
    dst = edge_index[1]
    dst32d = dst.reshape(NW, 40, 125)
    src16 = src.reshape(NS, 2, 40, 125)
    dst16 = dst.reshape(NS, 2, 40, 125)
    src32 = src.reshape(NW, 125, 40)
    dst32 = dst.reshape(NW, 125, 40)
    ones125 = jnp.ones((125,), jnp.float32)
    zeros1 = jnp.zeros((ROWS,), jnp.float32)
    zeros2 = jnp.zeros((ROWS, 128), jnp.float32)
    ea4 = edge_attr.reshape(NW, 125, 40, 16)

    p = _degree_sc(dst32d, ones125, zeros1).reshape(NC, NP)
    pt = p.T

    nblk = NP // 256
    row_spec = pl.BlockSpec((256, 512), lambda i: (i, 0))
    chunk_spec = pl.BlockSpec((256, 128), lambda i: (i, 0))
    pt_spec = pl.BlockSpec((256, 2), lambda i: (i, 0))

    h1, hs0, hs1, hs2, hs3 = pl.pallas_call(
        _mm1_tc,
        grid=(nblk,),
        in_specs=[
            pl.BlockSpec((256, DN), lambda i: (i, 0)),
            pl.BlockSpec((DN, H), lambda i: (0, 0)),
            pt_spec,
        ],
        out_specs=[row_spec, chunk_spec, chunk_spec, chunk_spec, chunk_spec],
        out_shape=[_f32(NP, H)] + [_f32(NP, 128)] * 4,
    )(x, W1, pt)

    g0, g1, g2, g3 = _agg_sc(hs0, hs1, hs2, hs3, src16, dst16, zeros2)

    h2, q0, q1, q2, q3 = pl.pallas_call(
        _layer2_tc,
        grid=(nblk,),
        in_specs=[
            chunk_spec, chunk_spec, chunk_spec, chunk_spec,
            row_spec,
            pt_spec,
            pl.BlockSpec((1, H), lambda i: (0, 0)),
            pl.BlockSpec((H, H), lambda i: (0, 0)),
        ],
        out_specs=[row_spec, chunk_spec, chunk_spec, chunk_spec, chunk_spec],
        out_shape=[_f32(NP, H)] + [_f32(NP, 128)] * 4,
    )(g0, g1, g2, g3, h1, pt, b1.reshape(1, H), W2)

    r0, r1, r2, r3 = _agg_sc(q0, q1, q2, q3, src16, dst16, zeros2)

    A, B = pl.pallas_call(
        _tables_tc,
        grid=(nblk,),
        in_specs=[
            chunk_spec, chunk_spec, chunk_spec, chunk_spec,
            row_spec,
            pt_spec,
            pl.BlockSpec((1, H), lambda i: (0, 0)),
            pl.BlockSpec((H, H), lambda i: (0, 0)),
            pl.BlockSpec((H, H), lambda i: (0, 0)),
        ],
        out_specs=[row_spec, row_spec],
        out_shape=[_f32(NP, H), _f32(NP, H)],
    )(r0, r1, r2, r3, h2, pt, b2.reshape(1, H), Wm1[0:H, :], Wm1[H:2 * H, :])

    Gh = [_edge_gather_sc(0, A, B, src32, dst32),
          _edge_gather_sc(1, A, B, src32, dst32)]

    def edge_mlp_half(g_lo, cnt, G):
        g4 = G.reshape(NW, cnt, 40, H)
        return pl.pallas_call(
            _edge_mlp_tc,
            grid=(NW, cnt),
            in_specs=[
                pl.BlockSpec((1, 1, 40, H), lambda w, j: (w, j, 0, 0)),
                pl.BlockSpec((1, 1, 40, 16),
                             lambda w, j, g_lo=g_lo: (w, g_lo + j, 0, 0)),
                pl.BlockSpec((16, H), lambda w, j: (0, 0)),
                pl.BlockSpec((1, H), lambda w, j: (0, 0)),
                pl.BlockSpec((H, 1), lambda w, j: (0, 0)),
                pl.BlockSpec((1, 1), lambda w, j: (0, 0)),
            ],
            out_specs=pl.BlockSpec((1, 1, 40, 1), lambda w, j: (w, j, 0, 0)),
            out_shape=_f32(NW, cnt, 40, 1),
        )(g4, ea4, Wm1[2 * H:, :], bm1.reshape(1, H), Wm2, bm2.reshape(1, 1))

    o0 = edge_mlp_half(0, 64, Gh[0])
    o1 = edge_mlp_half(64, 61, Gh[1])
    out = jnp.concatenate([o0[..., 0], o1[..., 0]], axis=1)
    return out.reshape(E)


# speedup vs baseline: 2.4645x; 2.4645x over previous
import functools

import jax
import jax.numpy as jnp
from jax import lax
from jax.experimental import pallas as pl
from jax.experimental.pallas import tpu as pltpu
from jax.experimental.pallas import tpu_sc as plsc

N = 10000
NP = 10240
E = 160000
DN = 256
H = 512
NC = 2
NS = 16
NW = NC * NS
EW = E // NW
ES = E // NS
ROWS = NP // NS

@functools.lru_cache(maxsize=None)
def _sc_kernels():
    mesh = plsc.VectorSubcoreMesh(
        core_axis_name="c", subcore_axis_name="s", num_cores=NC, num_subcores=NS
    )

    @functools.partial(
        pl.kernel,
        out_type=jax.ShapeDtypeStruct((NC * NP,), jnp.float32),
        mesh=mesh,
        scratch_types=[
            pltpu.VMEM((40, 125), jnp.int32),
            pltpu.VMEM((125,), jnp.float32),
            pltpu.VMEM_SHARED((NP,), jnp.float32),
        ],
    )
    def degree_k(dst_r, ones, zeros1, out, idx_v, ones_v, acc):
        c = lax.axis_index("c")
        s = lax.axis_index("s")
        wid = s * NC + c
        pltpu.sync_copy(zeros1, acc.at[pl.ds(s * ROWS, ROWS)])
        pltpu.sync_copy(dst_r.at[wid], idx_v)
        pltpu.sync_copy(ones, ones_v)
        plsc.subcore_barrier()

        @pl.loop(0, 40)
        def _(j):
            pltpu.sync_copy(ones_v, acc.at[idx_v.at[j]], add=True)

        plsc.subcore_barrier()
        pltpu.sync_copy(acc.at[pl.ds(s * ROWS, ROWS)],
                        out.at[pl.ds(c * NP + s * ROWS, ROWS)])


    @functools.partial(
        pl.kernel,
        out_type=[jax.ShapeDtypeStruct((NP, 128), jnp.float32) for _ in range(4)],
        mesh=mesh,
        scratch_types=[
            pltpu.VMEM((40, 125), jnp.int32),
            pltpu.VMEM((40, 125), jnp.int32),
            pltpu.VMEM((125, 128), jnp.float32),
            pltpu.VMEM((125, 128), jnp.float32),
            pltpu.VMEM_SHARED((NP, 128), jnp.float32),
            pltpu.SemaphoreType.DMA,
            pltpu.SemaphoreType.DMA,
        ],
    )
    def agg_k(hs0, hs1, hs2, hs3, src16, dst16, zeros2, a0, a1, a2, a3,
                idxs, idxd, buf0, buf1, acc, sem0, sem1):
        c = lax.axis_index("c")
        s = lax.axis_index("s")

        def do_chunk(hs, agg):
            pltpu.sync_copy(zeros2, acc.at[pl.ds(s * ROWS, ROWS)])
            plsc.subcore_barrier()
            for half in range(2):
                pltpu.sync_copy(src16.at[s, half], idxs)
                pltpu.sync_copy(dst16.at[s, half], idxd)
                pltpu.async_copy(hs.at[idxs.at[0]], buf0, sem0)

                @pl.loop(0, 20)
                def _(t):
                    j0 = t * 2
                    pltpu.make_async_copy(hs.at[idxs.at[j0]], buf0, sem0).wait()
                    pltpu.async_copy(hs.at[idxs.at[j0 + 1]], buf1, sem1)
                    pltpu.sync_copy(buf0, acc.at[idxd.at[j0]], add=True)
                    pltpu.make_async_copy(hs.at[idxs.at[j0 + 1]], buf1,
                                          sem1).wait()

                    @pl.when(t < 19)
                    def _():
                        pltpu.async_copy(hs.at[idxs.at[j0 + 2]], buf0, sem0)

                    pltpu.sync_copy(buf1, acc.at[idxd.at[j0 + 1]], add=True)

            plsc.subcore_barrier()
            pltpu.sync_copy(acc.at[pl.ds(s * ROWS, ROWS)],
                            agg.at[pl.ds(s * ROWS, ROWS)])

        @pl.when(c == 0)
        def _():
            do_chunk(hs0, a0)
            do_chunk(hs1, a1)

        @pl.when(c == 1)
        def _():
            do_chunk(hs2, a2)
            do_chunk(hs3, a3)


    def make_edge_gather(g_lo, g_hi):
        cnt = g_hi - g_lo
        npairs = cnt // 2

        @functools.partial(
            pl.kernel,
            out_type=jax.ShapeDtypeStruct((NW * cnt * 40, H), jnp.float32),
            mesh=mesh,
            scratch_types=[
                pltpu.VMEM((125, 40), jnp.int32),
                pltpu.VMEM((125, 40), jnp.int32),
                pltpu.VMEM((2, 40, H), jnp.float32),
                pltpu.VMEM((2, 40, H), jnp.float32),
                pltpu.SemaphoreType.DMA,
                pltpu.SemaphoreType.DMA,
                pltpu.SemaphoreType.DMA,
                pltpu.SemaphoreType.DMA,
            ],
        )
        def edge_gather_k(A, B, src32, dst32, G, idxs, idxd, bufa, bufb,
                          sa0, sb0, sa1, sb1):
            c = lax.axis_index("c")
            s = lax.axis_index("s")
            wid = s * NC + c
            pltpu.sync_copy(src32.at[wid], idxs)
            pltpu.sync_copy(dst32.at[wid], idxd)
            obase = wid * cnt * 40

            def fire(j, p, sa, sb):
                pltpu.async_copy(A.at[idxs.at[j]], bufa.at[p], sa)
                pltpu.async_copy(B.at[idxd.at[j]], bufb.at[p], sb)

            def drain(j, p, sa, sb):
                pltpu.make_async_copy(A.at[idxs.at[j]], bufa.at[p], sa).wait()
                pltpu.make_async_copy(B.at[idxd.at[j]], bufb.at[p], sb).wait()

            def addout(j, p):
                @pl.loop(0, 40)
                def _(r):
                    for k in range(H // 16):
                        sl = pl.ds(k * 16, 16)
                        bufa[p, r, sl] = bufa[p, r, sl] + bufb[p, r, sl]

                pltpu.sync_copy(bufa.at[p],
                                G.at[pl.ds(obase + (j - g_lo) * 40, 40)])

            fire(g_lo, 0, sa0, sb0)

            @pl.loop(0, npairs)
            def _(t):
                j0 = g_lo + t * 2
                drain(j0, 0, sa0, sb0)
                fire(j0 + 1, 1, sa1, sb1)
                addout(j0, 0)
                drain(j0 + 1, 1, sa1, sb1)

                @pl.when(j0 + 2 < g_hi)
                def _():
                    fire(j0 + 2, 0, sa0, sb0)

                addout(j0 + 1, 1)

            if cnt % 2:
                drain(g_hi - 1, 0, sa0, sb0)
                addout(g_hi - 1, 0)

        return edge_gather_k

    return degree_k, agg_k, make_edge_gather(0, 64), make_edge_gather(64, 125)


def _degree_sc(*args):
    return _sc_kernels()[0](*args)


def _agg_sc(*args):
    return _sc_kernels()[1](*args)


def _edge_gather_sc(half, *args):
    return _sc_kernels()[2 + half](*args)


def _mm1_tc(x_ref, w_ref, pt_ref, h_ref, o0, o1, o2, o3):
    h = jnp.dot(x_ref[...], w_ref[...], preferred_element_type=jnp.float32)
    h_ref[...] = h
    deg = 1.0 + pt_ref[:, 0:1] + pt_ref[:, 1:2]
    hs = h * lax.rsqrt(deg)
    o0[...] = hs[:, 0:128]
    o1[...] = hs[:, 128:256]
    o2[...] = hs[:, 256:384]
    o3[...] = hs[:, 384:512]


def _layer2_tc(a0, a1, a2, a3, h1_ref, pt_ref, b1_ref, w2_ref,
               h2_ref, q0, q1, q2, q3):
    agg = jnp.concatenate([a0[...], a1[...], a2[...], a3[...]], axis=1)
    deg = 1.0 + pt_ref[:, 0:1] + pt_ref[:, 1:2]
    dinv = lax.rsqrt(deg)
    z1 = jnp.maximum(agg * dinv + h1_ref[...] * (1.0 / deg) + b1_ref[...], 0.0)
    h2 = jnp.dot(z1, w2_ref[...], preferred_element_type=jnp.float32)
    h2_ref[...] = h2
    hs = h2 * dinv
    q0[...] = hs[:, 0:128]
    q1[...] = hs[:, 128:256]
    q2[...] = hs[:, 256:384]
    q3[...] = hs[:, 384:512]


def _tables_tc(a0, a1, a2, a3, h2_ref, pt_ref, b2_ref, wa_ref, wb_ref,
               A_ref, B_ref):
    agg = jnp.concatenate([a0[...], a1[...], a2[...], a3[...]], axis=1)
    deg = 1.0 + pt_ref[:, 0:1] + pt_ref[:, 1:2]
    z2 = agg * lax.rsqrt(deg) + h2_ref[...] * (1.0 / deg) + b2_ref[...]
    A_ref[...] = jnp.dot(z2, wa_ref[...], preferred_element_type=jnp.float32)
    B_ref[...] = jnp.dot(z2, wb_ref[...], preferred_element_type=jnp.float32)


def _edge_mlp_tc(g_ref, ea_ref, wme_ref, bm1_ref, wm2_ref, bm2_ref, out_ref):
    cc = jnp.dot(ea_ref[...], wme_ref[...], preferred_element_type=jnp.float32)
    t = jnp.maximum(g_ref[...] + cc + bm1_ref[...], 0.0)
    out_ref[...] = (
        jnp.dot(t, wm2_ref[...], preferred_element_type=jnp.float32)
        + bm2_ref[...]
    )


def _f32(*shape):
    return jax.ShapeDtypeStruct(shape, jnp.float32)


def kernel(x, edge_index, edge_attr, W1, b1, W2, b2, Wm1, bm1, Wm2, bm2):
    src = edge_index[0]
    dst = edge_index[1]
    dst32d = dst.reshape(NW, 40, 125)
    src16 = src.reshape(NS, 2, 40, 125)
    dst16 = dst.reshape(NS, 2, 40, 125)
    src32 = src.reshape(NW, 125, 40)
    dst32 = dst.reshape(NW, 125, 40)
    ones125 = jnp.ones((125,), jnp.float32)
    zeros1 = jnp.zeros((ROWS,), jnp.float32)
    zeros2 = jnp.zeros((ROWS, 128), jnp.float32)
    ea4 = edge_attr.reshape(NW, 125, 40, 16)

    p = _degree_sc(dst32d, ones125, zeros1).reshape(NC, NP)
    pt = p.T

    nblk = NP // 256
    row_spec = pl.BlockSpec((256, 512), lambda i: (i, 0))
    chunk_spec = pl.BlockSpec((256, 128), lambda i: (i, 0))
    pt_spec = pl.BlockSpec((256, 2), lambda i: (i, 0))

    h1, hs0, hs1, hs2, hs3 = pl.pallas_call(
        _mm1_tc,
        grid=(nblk,),
        in_specs=[
            pl.BlockSpec((256, DN), lambda i: (i, 0)),
            pl.BlockSpec((DN, H), lambda i: (0, 0)),
            pt_spec,
        ],
        out_specs=[row_spec, chunk_spec, chunk_spec, chunk_spec, chunk_spec],
        out_shape=[_f32(NP, H)] + [_f32(NP, 128)] * 4,
    )(x, W1, pt)

    g0, g1, g2, g3 = _agg_sc(hs0, hs1, hs2, hs3, src16, dst16, zeros2)

    h2, q0, q1, q2, q3 = pl.pallas_call(
        _layer2_tc,
        grid=(nblk,),
        in_specs=[
            chunk_spec, chunk_spec, chunk_spec, chunk_spec,
            row_spec,
            pt_spec,
            pl.BlockSpec((1, H), lambda i: (0, 0)),
            pl.BlockSpec((H, H), lambda i: (0, 0)),
        ],
        out_specs=[row_spec, chunk_spec, chunk_spec, chunk_spec, chunk_spec],
        out_shape=[_f32(NP, H)] + [_f32(NP, 128)] * 4,
    )(g0, g1, g2, g3, h1, pt, b1.reshape(1, H), W2)

    r0, r1, r2, r3 = _agg_sc(q0, q1, q2, q3, src16, dst16, zeros2)

    A, B = pl.pallas_call(
        _tables_tc,
        grid=(nblk,),
        in_specs=[
            chunk_spec, chunk_spec, chunk_spec, chunk_spec,
            row_spec,
            pt_spec,
            pl.BlockSpec((1, H), lambda i: (0, 0)),
            pl.BlockSpec((H, H), lambda i: (0, 0)),
            pl.BlockSpec((H, H), lambda i: (0, 0)),
        ],
        out_specs=[row_spec, row_spec],
        out_shape=[_f32(NP, H), _f32(NP, H)],
    )(r0, r1, r2, r3, h2, pt, b2.reshape(1, H), Wm1[0:H, :], Wm1[H:2 * H, :])

    Gh = [_edge_gather_sc(0, A, B, src32, dst32),
          _edge_gather_sc(1, A, B, src32, dst32)]

    def edge_mlp_half(g_lo, cnt, G):
        rows = NW * cnt * 40
        ea_h = ea4[:, g_lo:g_lo + cnt].reshape(rows, 16)
        eb = 640
        return pl.pallas_call(
            _edge_mlp_tc,
            grid=(rows // eb,),
            in_specs=[
                pl.BlockSpec((eb, H), lambda i: (i, 0)),
                pl.BlockSpec((eb, 16), lambda i: (i, 0)),
                pl.BlockSpec((16, H), lambda i: (0, 0)),
                pl.BlockSpec((1, H), lambda i: (0, 0)),
                pl.BlockSpec((H, 1), lambda i: (0, 0)),
                pl.BlockSpec((1, 1), lambda i: (0, 0)),
            ],
            out_specs=pl.BlockSpec((eb, 1), lambda i: (i, 0)),
            out_shape=_f32(rows, 1),
        )(G, ea_h, Wm1[2 * H:, :], bm1.reshape(1, H), Wm2, bm2.reshape(1, 1))

    o0 = edge_mlp_half(0, 64, Gh[0])
    o1 = edge_mlp_half(64, 61, Gh[1])
    out = jnp.concatenate([o0.reshape(NW, 64 * 40), o1.reshape(NW, 61 * 40)],
                          axis=1)
    return out.reshape(E)
